# baseline (speedup 1.0000x reference)
import functools

import jax
import jax.numpy as jnp
from jax import lax
from jax.experimental import pallas as pl
from jax.experimental.pallas import tpu as pltpu
from jax.experimental.pallas import tpu_sc as plsc

N = 10000
NPAD = 10240
C = 128
KW = 256
NSAMP = 16
G = 16
W = NPAD // G
RB = 128
GRID = NPAD // RB
S = 4
CHR = NPAD // S
GRIDC = CHR // RB
CAND = NSAMP * G
BIGF = 1e30


def _lnorm(x, g, b, eps=1e-5):
    mu = jnp.mean(x, axis=-1, keepdims=True)
    xm = x - mu
    var = jnp.mean(xm * xm, axis=-1, keepdims=True)
    return xm / jnp.sqrt(var + eps) * g + b


def _kqkv_body(pts_blk, feat, wq, bq_, wk, bk_, wv, bv_, q_out, k_out, v_out):
    xb = feat[...].astype(jnp.bfloat16)
    q_out[...] = jnp.dot(xb, wq[...].astype(jnp.bfloat16),
                         preferred_element_type=jnp.float32) + bq_[...]
    k_out[:, 0:C] = jnp.dot(xb, wk[...].astype(jnp.bfloat16),
                            preferred_element_type=jnp.float32) + bk_[...]
    k_out[:, C:C + 8] = pts_blk[...]
    k_out[:, C + 8:KW] = jnp.zeros((RB, KW - C - 8), jnp.float32)
    v_out[...] = jnp.dot(xb, wv[...].astype(jnp.bfloat16),
                         preferred_element_type=jnp.float32) + bv_[...]


def _kqkv(pts8, featpad, wq, bq_, wk, bk_, wv, bv_):
    full = lambda shape: pl.BlockSpec(shape, lambda i: (0, 0))
    return pl.pallas_call(
        _kqkv_body,
        grid=(GRID,),
        in_specs=[
            pl.BlockSpec((RB, 8), lambda i: (i, 0)),
            pl.BlockSpec((RB, C), lambda i: (i, 0)),
            full((C, C)), full((1, C)),
            full((C, C)), full((1, C)),
            full((C, C)), full((1, C)),
        ],
        out_specs=[
            pl.BlockSpec((RB, C), lambda i: (i, 0)),
            pl.BlockSpec((RB, KW), lambda i: (i, 0)),
            pl.BlockSpec((RB, C), lambda i: (i, 0)),
        ],
        out_shape=[
            jax.ShapeDtypeStruct((NPAD, C), jnp.float32),
            jax.ShapeDtypeStruct((NPAD, KW), jnp.float32),
            jax.ShapeDtypeStruct((NPAD, C), jnp.float32),
        ],
    )(pts8, featpad, wq, bq_, wk, bk_, wv, bv_)


def _kad_body(pts_blk, ptsTm2, sqrow, gid_out):
    xq = pts_blk[...].astype(jnp.bfloat16)
    ptm2 = ptsTm2[...].astype(jnp.bfloat16)
    e = sqrow[...] + jnp.dot(xq, ptm2, preferred_element_type=jnp.float32)

    dg = e[:, 0:W]
    for j in range(1, G):
        dg = jnp.minimum(dg, e[:, j * W:(j + 1) * W])

    iota = lax.broadcasted_iota(jnp.int32, (RB, W), 1).astype(jnp.float32)
    gids = []
    for _ in range(NSAMP):
        m = jnp.min(dg, axis=1, keepdims=True)
        gsel = jnp.min(jnp.where(dg == m, iota, BIGF), axis=1, keepdims=True)
        gids.append(gsel)
        dg = jnp.where(iota == gsel, BIGF, dg)
    gid_out[...] = jnp.concatenate(gids, axis=1).astype(jnp.int32)


def _kad(pts8c, ptsTm2, sqrow):
    full = lambda shape: pl.BlockSpec(shape, lambda i: (0, 0))
    return pl.pallas_call(
        _kad_body,
        grid=(GRIDC,),
        in_specs=[
            pl.BlockSpec((RB, 8), lambda i: (i, 0)),
            full((8, NPAD)),
            full((1, NPAD)),
        ],
        out_specs=pl.BlockSpec((RB, NSAMP), lambda i: (i, 0)),
        out_shape=jax.ShapeDtypeStruct((CHR, NSAMP), jnp.int32),
    )(pts8c, ptsTm2, sqrow)


def _kb(tab, idxf):
    B = idxf.shape[0]
    info = plsc.get_sparse_core_info()
    nw = info.num_cores * info.num_subcores
    bpw = B // nw
    ch = 256
    assert bpw % ch == 0
    nch = bpw // ch
    mesh = plsc.VectorSubcoreMesh(core_axis_name="c", subcore_axis_name="s")

    @functools.partial(
        pl.kernel, mesh=mesh,
        out_type=jax.ShapeDtypeStruct((B, C), jnp.float32),
        scratch_types=[
            pltpu.VMEM((ch,), jnp.int32),
            pltpu.VMEM((ch, C), jnp.float32),
            pltpu.SemaphoreType.DMA,
        ],
    )
    def kb(tab_h, idx_h, o_h, idx_v, buf, sem):
        wid = lax.axis_index("s") * info.num_cores + lax.axis_index("c")
        base = wid * bpw

        def body(t, carry):
            off = base + t * ch
            pltpu.sync_copy(idx_h.at[pl.ds(off, ch)], idx_v)
            pltpu.async_copy(tab_h.at[idx_v], buf, sem).wait()
            pltpu.sync_copy(buf, o_h.at[pl.ds(off, ch)])
            return carry

        lax.fori_loop(0, nch, body, 0)

    return kb(tab, idxf)


def _kc_body(cpk, pts_blk, sqcol, gid_blk, idx_out):
    blk = cpk[...]
    cx = jnp.concatenate([blk[:, s * C:s * C + G] for s in range(NSAMP)], axis=1)
    cy = jnp.concatenate([blk[:, s * C + G:s * C + 2 * G] for s in range(NSAMP)], axis=1)
    cz = jnp.concatenate([blk[:, s * C + 2 * G:s * C + 3 * G] for s in range(NSAMP)], axis=1)
    csq = jnp.concatenate([blk[:, s * C + 3 * G:s * C + 4 * G] for s in range(NSAMP)], axis=1)
    p = pts_blk[...]
    qx, qy, qz = p[:, 0:1], p[:, 1:2], p[:, 2:3]
    f32 = lambda t: t.astype(jnp.bfloat16).astype(jnp.float32)
    dotb = f32(qx) * f32(cx) + f32(qy) * f32(cy) + f32(qz) * f32(cz)
    d = sqcol[...] + csq - 2.0 * dotb

    gid = gid_blk[...].astype(jnp.float32)
    j16 = lax.broadcasted_iota(jnp.int32, (1, G), 1).astype(jnp.float32) * float(W)
    cols = jnp.concatenate([gid[:, s:s + 1] + j16 for s in range(NSAMP)], axis=1)

    idxs = []
    for _ in range(NSAMP):
        m = jnp.min(d, axis=1, keepdims=True)
        csel = jnp.min(jnp.where(d == m, cols, BIGF), axis=1, keepdims=True)
        idxs.append(csel)
        d = jnp.where(cols == csel, BIGF, d)
    idx_out[...] = jnp.concatenate(idxs, axis=1).astype(jnp.int32)


def _kc(cpk, pts8, sqcol, gid):
    return pl.pallas_call(
        _kc_body,
        grid=(GRIDC,),
        in_specs=[
            pl.BlockSpec((RB, NSAMP * C), lambda i: (i, 0)),
            pl.BlockSpec((RB, 8), lambda i: (i, 0)),
            pl.BlockSpec((RB, 1), lambda i: (i, 0)),
            pl.BlockSpec((RB, NSAMP), lambda i: (i, 0)),
        ],
        out_specs=pl.BlockSpec((RB, NSAMP), lambda i: (i, 0)),
        out_shape=jax.ShapeDtypeStruct((CHR, NSAMP), jnp.int32),
    )(cpk, pts8, sqcol, gid)


def _kd(ktab, vtab, idxf):
    B = idxf.shape[0]
    info = plsc.get_sparse_core_info()
    nw = info.num_cores * info.num_subcores
    bpw = B // nw
    ch = 256
    assert bpw % ch == 0
    nch = bpw // ch
    mesh = plsc.VectorSubcoreMesh(core_axis_name="c", subcore_axis_name="s")

    @functools.partial(
        pl.kernel, mesh=mesh,
        out_type=[
            jax.ShapeDtypeStruct((B, KW), jnp.float32),
            jax.ShapeDtypeStruct((B, C), jnp.float32),
        ],
        scratch_types=[
            pltpu.VMEM((ch,), jnp.int32),
            pltpu.VMEM((ch, KW), jnp.float32),
            pltpu.VMEM((ch, C), jnp.float32),
            pltpu.SemaphoreType.DMA,
        ],
    )
    def kd(kt, vt, idx_h, ok, ov, idx_v, bk_, bv_, sem):
        wid = lax.axis_index("s") * info.num_cores + lax.axis_index("c")
        base = wid * bpw

        def body(t, carry):
            off = base + t * ch
            pltpu.sync_copy(idx_h.at[pl.ds(off, ch)], idx_v)
            c1 = pltpu.async_copy(kt.at[idx_v], bk_, sem)
            c2 = pltpu.async_copy(vt.at[idx_v], bv_, sem)
            c1.wait()
            c2.wait()
            pltpu.sync_copy(bk_, ok.at[pl.ds(off, ch)])
            pltpu.sync_copy(bv_, ov.at[pl.ds(off, ch)])
            return carry

        lax.fori_loop(0, nch, body, 0)

    return kd(ktab, vtab, idxf)


def _ke_body(q, feat, pts, gkp, gv, wp, bp_, gpg, bpln, wg1, bg1_,
             gg_, bgln, wg2, bg2_, wo, bo_, out):
    bf = jnp.bfloat16
    gk3 = gkp[...].reshape(RB, NSAMP, KW)
    rel = gk3[:, :, C:C + 8] - pts[...].reshape(RB, 1, 8)
    pfeat = jnp.dot(rel.reshape(RB * NSAMP, 8).astype(bf), wp[...].astype(bf),
                    preferred_element_type=jnp.float32) + bp_[...]
    pfeat = jnp.maximum(_lnorm(pfeat, gpg[...], bpln[...]), 0.0)

    a = (q[...].reshape(RB, 1, C) - gk3[:, :, 0:C]
         + pfeat.reshape(RB, NSAMP, C))
    h = jnp.dot(a.reshape(RB * NSAMP, C).astype(bf), wg1[...].astype(bf),
                preferred_element_type=jnp.float32) + bg1_[...]
    h = jnp.maximum(_lnorm(h, gg_[...], bgln[...]), 0.0)
    h = jnp.dot(h.astype(bf), wg2[...].astype(bf),
                preferred_element_type=jnp.float32) + bg2_[...]
    h3 = h.reshape(RB, NSAMP, C)
    mx = jnp.max(h3, axis=1, keepdims=True)
    e = jnp.exp(h3 - mx)
    wgt = e / jnp.sum(e, axis=1, keepdims=True)

    vv = gv[...].reshape(RB, NSAMP, C) + pfeat.reshape(RB, NSAMP, C)
    t = jnp.sum(vv * wgt, axis=1)
    out[...] = (jnp.dot(t.astype(bf), wo[...].astype(bf),
                        preferred_element_type=jnp.float32)
                + bo_[...] + feat[...])


def _ke(q, featpad, pts8, gkp, gv, wp8, bp_, gpg, bpln, wg1, bg1_,
        gg_, bgln, wg2, bg2_, wo, bo_):
    full = lambda shape: pl.BlockSpec(shape, lambda i: (0, 0))
    return pl.pallas_call(
        _ke_body,
        grid=(GRIDC,),
        in_specs=[
            pl.BlockSpec((RB, C), lambda i: (i, 0)),
            pl.BlockSpec((RB, C), lambda i: (i, 0)),
            pl.BlockSpec((RB, 8), lambda i: (i, 0)),
            pl.BlockSpec((RB * NSAMP, KW), lambda i: (i, 0)),
            pl.BlockSpec((RB * NSAMP, C), lambda i: (i, 0)),
            full((8, C)), full((1, C)), full((1, C)), full((1, C)),
            full((C, C)), full((1, C)), full((1, C)), full((1, C)),
            full((C, C)), full((1, C)),
            full((C, C)), full((1, C)),
        ],
        out_specs=pl.BlockSpec((RB, C), lambda i: (i, 0)),
        out_shape=jax.ShapeDtypeStruct((CHR, C), jnp.float32),
    )(q, featpad, pts8, gkp, gv, wp8, bp_, gpg, bpln, wg1, bg1_,
      gg_, bgln, wg2, bg2_, wo, bo_)


def kernel(points, features, Wq, bq, Wk, bk, Wv, bv, Wp, bp, gp, bp_ln,
           Wg1, bg1, gg, bg_ln, Wg2, bg2, Wo, bo):
    f32 = jnp.float32
    pts8 = jnp.zeros((NPAD, 8), f32)
    pts8 = pts8.at[:N, :3].set(points).at[N:, :3].set(100.0)
    ptsTm2 = (-2.0 * pts8).T
    featpad = jnp.zeros((NPAD, C), f32).at[:N].set(features)
    sq = jnp.sum(points * points, axis=1)
    sqpad = jnp.full((NPAD,), 30000.0, f32).at[:N].set(sq)
    sqcol = sqpad.reshape(NPAD, 1)
    sqrow = sqpad.reshape(1, NPAD)

    row = lambda x: x.reshape(1, C)
    wp8 = jnp.zeros((8, C), f32).at[:3].set(Wp)

    q, kp, v = _kqkv(pts8, featpad, Wq, row(bq), Wk, row(bk), Wv, row(bv))

    tab = jnp.concatenate(
        [pts8[:, 0].reshape(G, W).T, pts8[:, 1].reshape(G, W).T,
         pts8[:, 2].reshape(G, W).T, sqpad.reshape(G, W).T,
         jnp.zeros((W, C - 4 * G), f32)], axis=1)


    outs = []
    for c in range(S):
        lo, hi = c * CHR, (c + 1) * CHR
        gid = _kad(pts8[lo:hi], ptsTm2, sqrow)
        cpk = _kb(tab, gid.reshape(-1))
        idx = _kc(cpk.reshape(CHR, NSAMP * C), pts8[lo:hi], sqcol[lo:hi], gid)
        gkp, gv = _kd(kp, v, idx.reshape(-1))
        outs.append(_ke(q[lo:hi], featpad[lo:hi], pts8[lo:hi], gkp, gv,
                        wp8, row(bp), row(gp), row(bp_ln), Wg1, row(bg1),
                        row(gg), row(bg_ln), Wg2, row(bg2), Wo, row(bo)))
    return jnp.concatenate(outs, axis=0)[:N]


# speedup vs baseline: 1.0067x; 1.0067x over previous
import functools

import jax
import jax.numpy as jnp
from jax import lax
from jax.experimental import pallas as pl
from jax.experimental.pallas import tpu as pltpu
from jax.experimental.pallas import tpu_sc as plsc

N = 10000
NPAD = 10240
C = 128
KW = 256
NSAMP = 16
G = 16
W = NPAD // G
RB = 128
GRID = NPAD // RB
S = 2
CHR = NPAD // S
GRIDC = CHR // RB
CAND = NSAMP * G
BIGF = 1e30


def _lnorm(x, g, b, eps=1e-5):
    mu = jnp.mean(x, axis=-1, keepdims=True)
    xm = x - mu
    var = jnp.mean(xm * xm, axis=-1, keepdims=True)
    return xm / jnp.sqrt(var + eps) * g + b


def _kqkv_body(pts_blk, feat, wq, bq_, wk, bk_, wv, bv_, q_out, k_out, v_out):
    xb = feat[...].astype(jnp.bfloat16)
    q_out[...] = jnp.dot(xb, wq[...].astype(jnp.bfloat16),
                         preferred_element_type=jnp.float32) + bq_[...]
    k_out[:, 0:C] = jnp.dot(xb, wk[...].astype(jnp.bfloat16),
                            preferred_element_type=jnp.float32) + bk_[...]
    k_out[:, C:C + 8] = pts_blk[...]
    k_out[:, C + 8:KW] = jnp.zeros((RB, KW - C - 8), jnp.float32)
    v_out[...] = jnp.dot(xb, wv[...].astype(jnp.bfloat16),
                         preferred_element_type=jnp.float32) + bv_[...]


def _kqkv(pts8, featpad, wq, bq_, wk, bk_, wv, bv_):
    full = lambda shape: pl.BlockSpec(shape, lambda i: (0, 0))
    return pl.pallas_call(
        _kqkv_body,
        grid=(GRID,),
        in_specs=[
            pl.BlockSpec((RB, 8), lambda i: (i, 0)),
            pl.BlockSpec((RB, C), lambda i: (i, 0)),
            full((C, C)), full((1, C)),
            full((C, C)), full((1, C)),
            full((C, C)), full((1, C)),
        ],
        out_specs=[
            pl.BlockSpec((RB, C), lambda i: (i, 0)),
            pl.BlockSpec((RB, KW), lambda i: (i, 0)),
            pl.BlockSpec((RB, C), lambda i: (i, 0)),
        ],
        out_shape=[
            jax.ShapeDtypeStruct((NPAD, C), jnp.float32),
            jax.ShapeDtypeStruct((NPAD, KW), jnp.float32),
            jax.ShapeDtypeStruct((NPAD, C), jnp.float32),
        ],
    )(pts8, featpad, wq, bq_, wk, bk_, wv, bv_)


def _kad_body(pts_blk, ptsTm2, sqrow, gid_out):
    xq = pts_blk[...].astype(jnp.bfloat16)
    ptm2 = ptsTm2[...].astype(jnp.bfloat16)
    e = sqrow[...] + jnp.dot(xq, ptm2, preferred_element_type=jnp.float32)

    dg = e[:, 0:W]
    for j in range(1, G):
        dg = jnp.minimum(dg, e[:, j * W:(j + 1) * W])

    iota = lax.broadcasted_iota(jnp.int32, (RB, W), 1).astype(jnp.float32)
    gids = []
    for _ in range(NSAMP):
        m = jnp.min(dg, axis=1, keepdims=True)
        gsel = jnp.min(jnp.where(dg == m, iota, BIGF), axis=1, keepdims=True)
        gids.append(gsel)
        dg = jnp.where(iota == gsel, BIGF, dg)
    gid_out[...] = jnp.concatenate(gids, axis=1).astype(jnp.int32)


def _kad(pts8c, ptsTm2, sqrow):
    full = lambda shape: pl.BlockSpec(shape, lambda i: (0, 0))
    return pl.pallas_call(
        _kad_body,
        grid=(GRIDC,),
        in_specs=[
            pl.BlockSpec((RB, 8), lambda i: (i, 0)),
            full((8, NPAD)),
            full((1, NPAD)),
        ],
        out_specs=pl.BlockSpec((RB, NSAMP), lambda i: (i, 0)),
        out_shape=jax.ShapeDtypeStruct((CHR, NSAMP), jnp.int32),
    )(pts8c, ptsTm2, sqrow)


def _kb(tab, idxf):
    B = idxf.shape[0]
    info = plsc.get_sparse_core_info()
    nw = info.num_cores * info.num_subcores
    bpw = B // nw
    ch = 256
    assert bpw % ch == 0
    nch = bpw // ch
    mesh = plsc.VectorSubcoreMesh(core_axis_name="c", subcore_axis_name="s")

    @functools.partial(
        pl.kernel, mesh=mesh,
        out_type=jax.ShapeDtypeStruct((B, C), jnp.float32),
        scratch_types=[
            pltpu.VMEM((ch,), jnp.int32),
            pltpu.VMEM((ch, C), jnp.float32),
            pltpu.SemaphoreType.DMA,
        ],
    )
    def kb(tab_h, idx_h, o_h, idx_v, buf, sem):
        wid = lax.axis_index("s") * info.num_cores + lax.axis_index("c")
        base = wid * bpw

        def body(t, carry):
            off = base + t * ch
            pltpu.sync_copy(idx_h.at[pl.ds(off, ch)], idx_v)
            pltpu.async_copy(tab_h.at[idx_v], buf, sem).wait()
            pltpu.sync_copy(buf, o_h.at[pl.ds(off, ch)])
            return carry

        lax.fori_loop(0, nch, body, 0)

    return kb(tab, idxf)


def _kc_body(cpk, pts_blk, sqcol, gid_blk, idx_out):
    blk = cpk[...]
    cx = jnp.concatenate([blk[:, s * C:s * C + G] for s in range(NSAMP)], axis=1)
    cy = jnp.concatenate([blk[:, s * C + G:s * C + 2 * G] for s in range(NSAMP)], axis=1)
    cz = jnp.concatenate([blk[:, s * C + 2 * G:s * C + 3 * G] for s in range(NSAMP)], axis=1)
    csq = jnp.concatenate([blk[:, s * C + 3 * G:s * C + 4 * G] for s in range(NSAMP)], axis=1)
    p = pts_blk[...]
    qx, qy, qz = p[:, 0:1], p[:, 1:2], p[:, 2:3]
    f32 = lambda t: t.astype(jnp.bfloat16).astype(jnp.float32)
    dotb = f32(qx) * f32(cx) + f32(qy) * f32(cy) + f32(qz) * f32(cz)
    d = sqcol[...] + csq - 2.0 * dotb

    gid = gid_blk[...].astype(jnp.float32)
    j16 = lax.broadcasted_iota(jnp.int32, (1, G), 1).astype(jnp.float32) * float(W)
    cols = jnp.concatenate([gid[:, s:s + 1] + j16 for s in range(NSAMP)], axis=1)

    idxs = []
    for _ in range(NSAMP):
        m = jnp.min(d, axis=1, keepdims=True)
        csel = jnp.min(jnp.where(d == m, cols, BIGF), axis=1, keepdims=True)
        idxs.append(csel)
        d = jnp.where(cols == csel, BIGF, d)
    idx_out[...] = jnp.concatenate(idxs, axis=1).astype(jnp.int32)


def _kc(cpk, pts8, sqcol, gid):
    return pl.pallas_call(
        _kc_body,
        grid=(GRIDC,),
        in_specs=[
            pl.BlockSpec((RB, NSAMP * C), lambda i: (i, 0)),
            pl.BlockSpec((RB, 8), lambda i: (i, 0)),
            pl.BlockSpec((RB, 1), lambda i: (i, 0)),
            pl.BlockSpec((RB, NSAMP), lambda i: (i, 0)),
        ],
        out_specs=pl.BlockSpec((RB, NSAMP), lambda i: (i, 0)),
        out_shape=jax.ShapeDtypeStruct((CHR, NSAMP), jnp.int32),
    )(cpk, pts8, sqcol, gid)


def _kd(ktab, vtab, idxf):
    B = idxf.shape[0]
    info = plsc.get_sparse_core_info()
    nw = info.num_cores * info.num_subcores
    bpw = B // nw
    ch = 256
    assert bpw % ch == 0
    nch = bpw // ch
    mesh = plsc.VectorSubcoreMesh(core_axis_name="c", subcore_axis_name="s")

    @functools.partial(
        pl.kernel, mesh=mesh,
        out_type=[
            jax.ShapeDtypeStruct((B, KW), jnp.float32),
            jax.ShapeDtypeStruct((B, C), jnp.float32),
        ],
        scratch_types=[
            pltpu.VMEM((ch,), jnp.int32),
            pltpu.VMEM((ch, KW), jnp.float32),
            pltpu.VMEM((ch, C), jnp.float32),
            pltpu.SemaphoreType.DMA,
        ],
    )
    def kd(kt, vt, idx_h, ok, ov, idx_v, bk_, bv_, sem):
        wid = lax.axis_index("s") * info.num_cores + lax.axis_index("c")
        base = wid * bpw

        def body(t, carry):
            off = base + t * ch
            pltpu.sync_copy(idx_h.at[pl.ds(off, ch)], idx_v)
            c1 = pltpu.async_copy(kt.at[idx_v], bk_, sem)
            c2 = pltpu.async_copy(vt.at[idx_v], bv_, sem)
            c1.wait()
            c2.wait()
            pltpu.sync_copy(bk_, ok.at[pl.ds(off, ch)])
            pltpu.sync_copy(bv_, ov.at[pl.ds(off, ch)])
            return carry

        lax.fori_loop(0, nch, body, 0)

    return kd(ktab, vtab, idxf)


def _ke_body(q, feat, pts, gkp, gv, wp, bp_, gpg, bpln, wg1, bg1_,
             gg_, bgln, wg2, bg2_, wo, bo_, out):
    bf = jnp.bfloat16
    gk3 = gkp[...].reshape(RB, NSAMP, KW)
    rel = gk3[:, :, C:C + 8] - pts[...].reshape(RB, 1, 8)
    pfeat = jnp.dot(rel.reshape(RB * NSAMP, 8).astype(bf), wp[...].astype(bf),
                    preferred_element_type=jnp.float32) + bp_[...]
    pfeat = jnp.maximum(_lnorm(pfeat, gpg[...], bpln[...]), 0.0)

    a = (q[...].reshape(RB, 1, C) - gk3[:, :, 0:C]
         + pfeat.reshape(RB, NSAMP, C))
    h = jnp.dot(a.reshape(RB * NSAMP, C).astype(bf), wg1[...].astype(bf),
                preferred_element_type=jnp.float32) + bg1_[...]
    h = jnp.maximum(_lnorm(h, gg_[...], bgln[...]), 0.0)
    h = jnp.dot(h.astype(bf), wg2[...].astype(bf),
                preferred_element_type=jnp.float32) + bg2_[...]
    h3 = h.reshape(RB, NSAMP, C)
    mx = jnp.max(h3, axis=1, keepdims=True)
    e = jnp.exp(h3 - mx)
    wgt = e / jnp.sum(e, axis=1, keepdims=True)

    vv = gv[...].reshape(RB, NSAMP, C) + pfeat.reshape(RB, NSAMP, C)
    t = jnp.sum(vv * wgt, axis=1)
    out[...] = (jnp.dot(t.astype(bf), wo[...].astype(bf),
                        preferred_element_type=jnp.float32)
                + bo_[...] + feat[...])


def _ke(q, featpad, pts8, gkp, gv, wp8, bp_, gpg, bpln, wg1, bg1_,
        gg_, bgln, wg2, bg2_, wo, bo_):
    full = lambda shape: pl.BlockSpec(shape, lambda i: (0, 0))
    return pl.pallas_call(
        _ke_body,
        grid=(GRIDC,),
        in_specs=[
            pl.BlockSpec((RB, C), lambda i: (i, 0)),
            pl.BlockSpec((RB, C), lambda i: (i, 0)),
            pl.BlockSpec((RB, 8), lambda i: (i, 0)),
            pl.BlockSpec((RB * NSAMP, KW), lambda i: (i, 0)),
            pl.BlockSpec((RB * NSAMP, C), lambda i: (i, 0)),
            full((8, C)), full((1, C)), full((1, C)), full((1, C)),
            full((C, C)), full((1, C)), full((1, C)), full((1, C)),
            full((C, C)), full((1, C)),
            full((C, C)), full((1, C)),
        ],
        out_specs=pl.BlockSpec((RB, C), lambda i: (i, 0)),
        out_shape=jax.ShapeDtypeStruct((CHR, C), jnp.float32),
    )(q, featpad, pts8, gkp, gv, wp8, bp_, gpg, bpln, wg1, bg1_,
      gg_, bgln, wg2, bg2_, wo, bo_)


def kernel(points, features, Wq, bq, Wk, bk, Wv, bv, Wp, bp, gp, bp_ln,
           Wg1, bg1, gg, bg_ln, Wg2, bg2, Wo, bo):
    f32 = jnp.float32
    pts8 = jnp.zeros((NPAD, 8), f32)
    pts8 = pts8.at[:N, :3].set(points).at[N:, :3].set(100.0)
    ptsTm2 = (-2.0 * pts8).T
    featpad = jnp.zeros((NPAD, C), f32).at[:N].set(features)
    sq = jnp.sum(points * points, axis=1)
    sqpad = jnp.full((NPAD,), 30000.0, f32).at[:N].set(sq)
    sqcol = sqpad.reshape(NPAD, 1)
    sqrow = sqpad.reshape(1, NPAD)

    row = lambda x: x.reshape(1, C)
    wp8 = jnp.zeros((8, C), f32).at[:3].set(Wp)

    q, kp, v = _kqkv(pts8, featpad, Wq, row(bq), Wk, row(bk), Wv, row(bv))

    tab = jnp.concatenate(
        [pts8[:, 0].reshape(G, W).T, pts8[:, 1].reshape(G, W).T,
         pts8[:, 2].reshape(G, W).T, sqpad.reshape(G, W).T,
         jnp.zeros((W, C - 4 * G), f32)], axis=1)


    outs = []
    for c in range(S):
        lo, hi = c * CHR, (c + 1) * CHR
        gid = _kad(pts8[lo:hi], ptsTm2, sqrow)
        cpk = _kb(tab, gid.reshape(-1))
        idx = _kc(cpk.reshape(CHR, NSAMP * C), pts8[lo:hi], sqcol[lo:hi], gid)
        gkp, gv = _kd(kp, v, idx.reshape(-1))
        outs.append(_ke(q[lo:hi], featpad[lo:hi], pts8[lo:hi], gkp, gv,
                        wp8, row(bp), row(gp), row(bp_ln), Wg1, row(bg1),
                        row(gg), row(bg_ln), Wg2, row(bg2), Wo, row(bo)))
    return jnp.concatenate(outs, axis=0)[:N]
